# baseline (speedup 1.0000x reference)
import functools

import jax
import jax.numpy as jnp
import numpy as np
from jax import lax
from jax.experimental import pallas as pl
from jax.experimental.pallas import tpu as pltpu
from jax.experimental.pallas import tpu_sc as plsc

N = 10000
E = 320000
D = 128

NC = 2
NS = 16
NW = NC * NS
CH = 64
TOT = 160
EPT = CH * TOT
E_PAD = NW * EPT
N_PAD = 10240
ROWS_PT = N_PAD // NS
NB = 5
PFG = 3
PFI = 4

_mesh = plsc.VectorSubcoreMesh(core_axis_name="c", subcore_axis_name="s")

_BCAST_DNUMS = lax.GatherDimensionNumbers(
    offset_dims=(), collapsed_slice_dims=(0,), start_index_map=(0,))


@functools.partial(
    pl.kernel,
    out_type=jax.ShapeDtypeStruct((2, NC, N_PAD, D), jnp.float32),
    mesh=_mesh,
    scratch_types=[
        pltpu.VMEM((NB, CH), jnp.int32),
        pltpu.VMEM((NB, CH), jnp.int32),
        pltpu.VMEM((NB, CH), jnp.float32),
        pltpu.VMEM((NB, CH, D), jnp.float32),
        pltpu.VMEM_SHARED((N_PAD, D), jnp.float32),
        pltpu.SemaphoreType.DMA((NB,)),
        pltpu.SemaphoreType.DMA((NB,)),
        pltpu.SemaphoreType.DMA((NB,)),
        pltpu.SemaphoreType.DMA((NB,)),
        pltpu.SemaphoreType.DMA((NB,)),
    ],
)
def _sc_spmm(x1_hbm, x2_hbm, row_hbm, col_hbm, val_hbm, out_hbm,
             colb, rowb, valb, rows4, agg_sh, gsem, ssem, csem, rsem, vsem):
    cid = lax.axis_index("c")
    sid = lax.axis_index("s")
    wid = sid * NC + cid
    r0 = sid * ROWS_PT

    zeros16 = jnp.zeros((16,), jnp.float32)

    def _zero_buf0(i, carry):
        for c8 in range(D // 16):
            rows4[0, i, pl.ds(c8 * 16, 16)] = zeros16
        return carry

    def _scale(b):
        def _g(g, carry):
            vals_g = valb[b, pl.ds(g * 16, 16)]
            for e16 in range(16):
                vv = lax.gather(
                    vals_g, jnp.full((16, 1), e16, jnp.int32),
                    _BCAST_DNUMS, (1,),
                    mode=lax.GatherScatterMode.PROMISE_IN_BOUNDS)
                e = g * 16 + e16
                for c8 in range(D // 16):
                    sl = rows4[b, e, pl.ds(c8 * 16, 16)]
                    rows4[b, e, pl.ds(c8 * 16, 16)] = sl * vv
            return carry

        lax.fori_loop(0, CH // 16, _g, 0)

    def _idx_load(jc, slot):
        pltpu.async_copy(col_hbm.at[wid, jc], colb.at[slot], csem.at[slot])
        pltpu.async_copy(row_hbm.at[wid, jc], rowb.at[slot], rsem.at[slot])
        pltpu.async_copy(val_hbm.at[wid, jc], valb.at[slot], vsem.at[slot])

    def _pipeline(x_hbm):
        for k in range(PFI):
            _idx_load(k, k)
        for k in range(PFG):
            pltpu.make_async_copy(
                col_hbm.at[wid, k], colb.at[k], csem.at[k]).wait()
            pltpu.async_copy(x_hbm.at[colb.at[k]], rows4.at[k], gsem.at[k])

        def _slot(j5, b, carry):
            j = j5 * NB + b
            b1 = (b + PFI) % NB
            b2 = (b + PFG) % NB
            pltpu.make_async_copy(
                x_hbm.at[colb.at[b]], rows4.at[b], gsem.at[b]).wait()
            pltpu.make_async_copy(
                row_hbm.at[wid, j], rowb.at[b], rsem.at[b]).wait()
            pltpu.make_async_copy(
                val_hbm.at[wid, j], valb.at[b], vsem.at[b]).wait()
            _scale(b)
            pltpu.async_copy(
                rows4.at[b], agg_sh.at[rowb.at[b]], ssem.at[b], add=True)

            @pl.when(j >= 1)
            def _():
                pltpu.make_async_copy(
                    rows4.at[b1], agg_sh.at[rowb.at[b1]],
                    ssem.at[b1]).wait()

            @pl.when(j + PFI < TOT)
            def _():
                _idx_load(j + PFI, b1)

            @pl.when(j + PFG < TOT)
            def _():
                pltpu.make_async_copy(
                    col_hbm.at[wid, j + PFG], colb.at[b2],
                    csem.at[b2]).wait()
                pltpu.async_copy(
                    x_hbm.at[colb.at[b2]], rows4.at[b2], gsem.at[b2])
            return carry

        def _group(j5, carry):
            for b in range(NB):
                _slot(j5, b, carry)
            return carry

        lax.fori_loop(0, TOT // NB, _group, 0)
        pltpu.make_async_copy(
            rows4.at[(TOT - 1) % NB], agg_sh.at[rowb.at[(TOT - 1) % NB]],
            ssem.at[(TOT - 1) % NB]).wait()

    def _zero_agg():
        lax.fori_loop(0, CH, _zero_buf0, 0)
        for k in range(ROWS_PT // CH):
            pltpu.async_copy(rows4.at[0],
                             agg_sh.at[pl.ds(r0 + k * CH, CH)],
                             gsem.at[k % NB])
        for k in range(ROWS_PT // CH):
            pltpu.make_async_copy(rows4.at[0],
                                  agg_sh.at[pl.ds(r0 + k * CH, CH)],
                                  gsem.at[k % NB]).wait()

    _zero_agg()
    plsc.subcore_barrier()

    for inp, x_hbm in enumerate((x1_hbm, x2_hbm)):
        _pipeline(x_hbm)
        plsc.subcore_barrier()

        pltpu.sync_copy(agg_sh.at[pl.ds(r0, ROWS_PT)],
                        out_hbm.at[inp, cid, pl.ds(r0, ROWS_PT)])
        if inp == 0:
            _zero_agg()
        plsc.subcore_barrier()


EB = 4
EBR = N_PAD // EB


def _epilogue_body(parts_ref, W_ref, bias_ref, prelu_ref, linW_ref, linb_ref,
                   out_ref):
    agg = parts_ref[0, 0] + parts_ref[0, 1]
    h = jnp.dot(agg, W_ref[...].T, preferred_element_type=jnp.float32)
    t = h + bias_ref[0][None, :]
    p = prelu_ref[0, 0]
    t = jnp.where(t >= 0, t, p * t)
    wsum = jnp.sum(linW_ref[...], axis=0)
    bsum = jnp.sum(linb_ref[0])
    z = jnp.sum(t * wsum[None, :], axis=1) + bsum
    out_ref[0, 0, 0, :] = z


def _epilogue(parts, W_fc, gcn_bias, prelu_w, lin_W, lin_b):
    return pl.pallas_call(
        _epilogue_body,
        grid=(2, EB),
        in_specs=[
            pl.BlockSpec((1, NC, EBR, D), lambda i, b: (i, 0, b, 0)),
            pl.BlockSpec((D, D), lambda i, b: (0, 0)),
            pl.BlockSpec((1, D), lambda i, b: (0, 0)),
            pl.BlockSpec((1, 1), lambda i, b: (0, 0),
                         memory_space=pltpu.SMEM),
            pl.BlockSpec((D, D), lambda i, b: (0, 0)),
            pl.BlockSpec((1, D), lambda i, b: (0, 0)),
        ],
        out_specs=pl.BlockSpec((1, 1, 1, EBR), lambda i, b: (i, b, 0, 0)),
        out_shape=jax.ShapeDtypeStruct((2, EB, 1, EBR), jnp.float32),
    )(parts, W_fc, gcn_bias, prelu_w, lin_W, lin_b)


def kernel(x_1, x_2, adj_indices, adj_values, W_fc, prelu_w, gcn_bias, lin_W,
           lin_b):
    row = adj_indices[0]
    col = adj_indices[1]
    pad = E_PAD - E
    spread = jnp.asarray((np.arange(pad, dtype=np.int32) * 37) % N)
    row_p = jnp.concatenate([row, spread])
    col_p = jnp.concatenate([col, spread])
    val_p = jnp.concatenate([adj_values, jnp.zeros((pad,), jnp.float32)])
    parts = _sc_spmm(x_1, x_2,
                     row_p.reshape(NW, TOT, CH),
                     col_p.reshape(NW, TOT, CH),
                     val_p.reshape(NW, TOT, CH))
    z = _epilogue(parts, W_fc, gcn_bias.reshape(1, D),
                  prelu_w.reshape(1, 1), lin_W, lin_b.reshape(1, D))
    return z.reshape(2, N_PAD)[:, :N].reshape(2 * N)


# speedup vs baseline: 1.0579x; 1.0579x over previous
import functools

import jax
import jax.numpy as jnp
import numpy as np
from jax import lax
from jax.experimental import pallas as pl
from jax.experimental.pallas import tpu as pltpu
from jax.experimental.pallas import tpu_sc as plsc

N = 10000
E = 320000
D = 128

NC = 2
NS = 16
NW = NC * NS
CH = 64
TOT = 160
EPT = CH * TOT
E_PAD = NW * EPT
N_PAD = 10240
ROWS_PT = N_PAD // NS
NB = 5
PFG = 3
PFI = 4

_mesh = plsc.VectorSubcoreMesh(core_axis_name="c", subcore_axis_name="s")

_BCAST_DNUMS = lax.GatherDimensionNumbers(
    offset_dims=(), collapsed_slice_dims=(0,), start_index_map=(0,))


@functools.partial(
    pl.kernel,
    out_type=jax.ShapeDtypeStruct((2, NC, N_PAD, D), jnp.float32),
    mesh=_mesh,
    scratch_types=[
        pltpu.VMEM((NB, CH), jnp.int32),
        pltpu.VMEM((NB, CH), jnp.int32),
        pltpu.VMEM((NB, CH), jnp.float32),
        pltpu.VMEM((NB, CH, D), jnp.float32),
        pltpu.VMEM_SHARED((N_PAD, D), jnp.float32),
        pltpu.SemaphoreType.DMA((NB,)),
        pltpu.SemaphoreType.DMA((NB,)),
        pltpu.SemaphoreType.DMA((NB,)),
        pltpu.SemaphoreType.DMA((NB,)),
        pltpu.SemaphoreType.DMA((NB,)),
    ],
)
def _sc_spmm(x1_hbm, x2_hbm, idx_hbm, val_hbm, tidx_hbm, tval_hbm, out_hbm,
             colb, rowb, valb, rows4, agg_sh, gsem, ssem, csem, rsem, vsem):
    cid = lax.axis_index("c")
    sid = lax.axis_index("s")
    wid = sid * NC + cid
    r0 = sid * ROWS_PT

    zeros16 = jnp.zeros((16,), jnp.float32)

    def _zero_buf0(i, carry):
        for c8 in range(D // 16):
            rows4[0, i, pl.ds(c8 * 16, 16)] = zeros16
        return carry

    def _scale(b):
        def _g(g, carry):
            vals_g = valb[b, pl.ds(g * 16, 16)]
            for e16 in range(16):
                vv = lax.gather(
                    vals_g, jnp.full((16, 1), e16, jnp.int32),
                    _BCAST_DNUMS, (1,),
                    mode=lax.GatherScatterMode.PROMISE_IN_BOUNDS)
                e = g * 16 + e16
                for c8 in range(D // 16):
                    sl = rows4[b, e, pl.ds(c8 * 16, 16)]
                    rows4[b, e, pl.ds(c8 * 16, 16)] = sl * vv
            return carry

        lax.fori_loop(0, CH // 16, _g, 0)

    def _idx_load(jc, slot):
        off = wid * EPT + jc * CH

        @pl.when(wid < NW - 1)
        def _():
            pltpu.async_copy(idx_hbm.at[1, pl.ds(off, CH)], colb.at[slot],
                             csem.at[slot])
            pltpu.async_copy(idx_hbm.at[0, pl.ds(off, CH)], rowb.at[slot],
                             rsem.at[slot])
            pltpu.async_copy(val_hbm.at[pl.ds(off, CH)], valb.at[slot],
                             vsem.at[slot])

        @pl.when(wid == NW - 1)
        def _():
            pltpu.async_copy(tidx_hbm.at[1, pl.ds(jc * CH, CH)],
                             colb.at[slot], csem.at[slot])
            pltpu.async_copy(tidx_hbm.at[0, pl.ds(jc * CH, CH)],
                             rowb.at[slot], rsem.at[slot])
            pltpu.async_copy(tval_hbm.at[pl.ds(jc * CH, CH)],
                             valb.at[slot], vsem.at[slot])

    def _wait_idx(jc, slot, which):
        if which == "col":
            pltpu.make_async_copy(tidx_hbm.at[1, pl.ds(jc * CH, CH)],
                                  colb.at[slot], csem.at[slot]).wait()
        elif which == "row":
            pltpu.make_async_copy(tidx_hbm.at[0, pl.ds(jc * CH, CH)],
                                  rowb.at[slot], rsem.at[slot]).wait()
        else:
            pltpu.make_async_copy(tval_hbm.at[pl.ds(jc * CH, CH)],
                                  valb.at[slot], vsem.at[slot]).wait()

    def _pipeline(x_hbm):
        for k in range(PFI):
            _idx_load(k, k)
        for k in range(PFG):
            _wait_idx(k, k, "col")
            pltpu.async_copy(x_hbm.at[colb.at[k]], rows4.at[k], gsem.at[k])

        def _slot(j5, b, carry):
            j = j5 * NB + b
            b1 = (b + PFI) % NB
            b2 = (b + PFG) % NB
            pltpu.make_async_copy(
                x_hbm.at[colb.at[b]], rows4.at[b], gsem.at[b]).wait()
            _wait_idx(j, b, "row")
            _wait_idx(j, b, "val")
            _scale(b)
            pltpu.async_copy(
                rows4.at[b], agg_sh.at[rowb.at[b]], ssem.at[b], add=True)

            @pl.when(j >= 1)
            def _():
                pltpu.make_async_copy(
                    rows4.at[b1], agg_sh.at[rowb.at[b1]],
                    ssem.at[b1]).wait()

            @pl.when(j + PFI < TOT)
            def _():
                _idx_load(j + PFI, b1)

            @pl.when(j + PFG < TOT)
            def _():
                _wait_idx(j + PFG, b2, "col")
                pltpu.async_copy(
                    x_hbm.at[colb.at[b2]], rows4.at[b2], gsem.at[b2])
            return carry

        def _group(j5, carry):
            for b in range(NB):
                _slot(j5, b, carry)
            return carry

        lax.fori_loop(0, TOT // NB, _group, 0)
        pltpu.make_async_copy(
            rows4.at[(TOT - 1) % NB], agg_sh.at[rowb.at[(TOT - 1) % NB]],
            ssem.at[(TOT - 1) % NB]).wait()

    def _zero_agg():
        lax.fori_loop(0, CH, _zero_buf0, 0)
        for k in range(ROWS_PT // CH):
            pltpu.async_copy(rows4.at[0],
                             agg_sh.at[pl.ds(r0 + k * CH, CH)],
                             gsem.at[k % NB])
        for k in range(ROWS_PT // CH):
            pltpu.make_async_copy(rows4.at[0],
                                  agg_sh.at[pl.ds(r0 + k * CH, CH)],
                                  gsem.at[k % NB]).wait()

    _zero_agg()
    plsc.subcore_barrier()

    for inp, x_hbm in enumerate((x1_hbm, x2_hbm)):
        _pipeline(x_hbm)
        plsc.subcore_barrier()

        pltpu.sync_copy(agg_sh.at[pl.ds(r0, ROWS_PT)],
                        out_hbm.at[inp, cid, pl.ds(r0, ROWS_PT)])
        if inp == 0:
            _zero_agg()
        plsc.subcore_barrier()


EB = 4
EBR = N_PAD // EB


def _epilogue_body(parts_ref, W_ref, bias_ref, prelu_ref, linW_ref, linb_ref,
                   out_ref):
    agg = parts_ref[0, 0] + parts_ref[0, 1]
    h = jnp.dot(agg, W_ref[...].T, preferred_element_type=jnp.float32)
    t = h + bias_ref[0][None, :]
    p = prelu_ref[0, 0]
    t = jnp.where(t >= 0, t, p * t)
    wsum = jnp.sum(linW_ref[...], axis=0)
    bsum = jnp.sum(linb_ref[0])
    z = jnp.sum(t * wsum[None, :], axis=1) + bsum
    out_ref[0, 0, 0, :] = z


def _epilogue(parts, W_fc, gcn_bias, prelu_w, lin_W, lin_b):
    return pl.pallas_call(
        _epilogue_body,
        grid=(2, EB),
        in_specs=[
            pl.BlockSpec((1, NC, EBR, D), lambda i, b: (i, 0, b, 0)),
            pl.BlockSpec((D, D), lambda i, b: (0, 0)),
            pl.BlockSpec((1, D), lambda i, b: (0, 0)),
            pl.BlockSpec((1, 1), lambda i, b: (0, 0),
                         memory_space=pltpu.SMEM),
            pl.BlockSpec((D, D), lambda i, b: (0, 0)),
            pl.BlockSpec((1, D), lambda i, b: (0, 0)),
        ],
        out_specs=pl.BlockSpec((1, 1, 1, EBR), lambda i, b: (i, b, 0, 0)),
        out_shape=jax.ShapeDtypeStruct((2, EB, 1, EBR), jnp.float32),
    )(parts, W_fc, gcn_bias, prelu_w, lin_W, lin_b)


def kernel(x_1, x_2, adj_indices, adj_values, W_fc, prelu_w, gcn_bias, lin_W,
           lin_b):
    pad = E_PAD - E
    spread = jnp.asarray((np.arange(pad, dtype=np.int32) * 37) % N)
    t0 = (NW - 1) * EPT
    tidx = jnp.concatenate(
        [adj_indices[:, t0:], jnp.stack([spread, spread])], axis=1)
    tval = jnp.concatenate(
        [adj_values[t0:], jnp.zeros((pad,), jnp.float32)])
    parts = _sc_spmm(x_1, x_2, adj_indices, adj_values, tidx, tval)
    z = _epilogue(parts, W_fc, gcn_bias.reshape(1, D),
                  prelu_w.reshape(1, 1), lin_W, lin_b.reshape(1, D))
    return z.reshape(2, N_PAD)[:, :N].reshape(2 * N)
